# baseline (speedup 1.0000x reference)
import functools
import math

import jax
import jax.numpy as jnp
from jax import lax
from jax.experimental import pallas as pl
from jax.experimental.pallas import tpu as pltpu
from jax.experimental.pallas import tpu_sc as plsc

_RB = 128
_SLICES = (2048, 2048, 2048, 1536, 512)


def _sc_gather_slice(x_slice, table):
    (b,) = x_slice.shape
    _, d = table.shape
    info = plsc.get_sparse_core_info()
    nc, ns = info.num_cores, info.num_subcores
    nw = nc * ns
    b_per_w = b // nw
    k = 8
    n_chunks = b_per_w // k
    nbuf = min(6, n_chunks)

    mesh = plsc.VectorSubcoreMesh(core_axis_name="c", subcore_axis_name="s")

    @functools.partial(
        pl.kernel,
        mesh=mesh,
        out_type=jax.ShapeDtypeStruct((b, d), jnp.float32),
        scratch_types=[
            pltpu.VMEM((b_per_w,), jnp.int32),
            *[pltpu.VMEM((k, d), jnp.float32) for _ in range(nbuf)],
            *[pltpu.SemaphoreType.DMA for _ in range(nbuf)],
            *[pltpu.SemaphoreType.DMA for _ in range(nbuf)],
        ],
    )
    def gather_kernel(idx_hbm, table_hbm, out_hbm, idx_v, *rest):
        bufs = rest[:nbuf]
        gsems = rest[nbuf : 2 * nbuf]
        ssems = rest[2 * nbuf :]
        wid = lax.axis_index("s") * nc + lax.axis_index("c")
        base = wid * b_per_w
        pltpu.sync_copy(idx_hbm.at[pl.ds(base, b_per_w)], idx_v)
        gcp = [None] * nbuf
        scp = [None] * nbuf
        for c in range(nbuf):
            gcp[c] = pltpu.async_copy(
                table_hbm.at[idx_v.at[pl.ds(c * k, k)]], bufs[c], gsems[c]
            )
        for c in range(n_chunks):
            s = c % nbuf
            gcp[s].wait()
            scp[s] = pltpu.async_copy(
                bufs[s], out_hbm.at[pl.ds(base + c * k, k)], ssems[s]
            )
            nx = c + nbuf
            if nx < n_chunks:
                scp[s].wait()
                gcp[s] = pltpu.async_copy(
                    table_hbm.at[idx_v.at[pl.ds(nx * k, k)]], bufs[s], gsems[s]
                )
        for c in range(max(0, n_chunks - nbuf), n_chunks):
            scp[c % nbuf].wait()

    return gather_kernel(x_slice, table)


def _neg_log(d):
    return -math.log(10000.0) / float(d)


def _pe_tables(d):
    nl = _neg_log(d)

    def body(p_ref, q_ref):
        col = lax.broadcasted_iota(jnp.int32, (1, d), 1)
        w = jnp.exp((col - (col % 2)).astype(jnp.float32) * nl)
        r = lax.broadcasted_iota(jnp.int32, (_RB, 1), 0).astype(jnp.float32)
        ang = r * w
        p_ref[...] = jnp.cos(ang)
        q_ref[...] = jnp.sin(ang)

    return pl.pallas_call(
        body,
        out_shape=(
            jax.ShapeDtypeStruct((_RB, d), jnp.float32),
            jax.ShapeDtypeStruct((_RB, d), jnp.float32),
        ),
    )()


def _tc_combine_slice(g, p, q, prev_out, row0, b_total):
    bs, d = g.shape
    steps = bs // _RB
    blk0 = row0 // _RB
    scale = math.sqrt(float(d))
    nl = _neg_log(d)

    def body(g_ref, p_ref, q_ref, _prev_ref, o_ref):
        i = pl.program_id(0)
        col = lax.broadcasted_iota(jnp.int32, (1, d), 1)
        w = jnp.exp((col - (col % 2)).astype(jnp.float32) * nl)
        base_ang = ((blk0 + i) * _RB).astype(jnp.float32) * w
        sb = jnp.sin(base_ang)
        cb = jnp.cos(base_ang)
        even = (col % 2) == 0
        a = jnp.where(even, sb, cb)
        bv = jnp.where(even, cb, -sb)
        o_ref[...] = g_ref[...] * scale + a * p_ref[...] + bv * q_ref[...]

    kwargs = {}
    ins = [g, p, q]
    in_specs = [
        pl.BlockSpec((_RB, d), lambda i: (i, 0)),
        pl.BlockSpec((_RB, d), lambda i: (0, 0)),
        pl.BlockSpec((_RB, d), lambda i: (0, 0)),
    ]
    if prev_out is None:
        def body0(g_ref, p_ref, q_ref, o_ref):
            return body(g_ref, p_ref, q_ref, None, o_ref)
        fn = body0
    else:
        ins.append(prev_out)
        in_specs.append(pl.BlockSpec(memory_space=pltpu.HBM))
        kwargs["input_output_aliases"] = {3: 0}
        fn = body

    return pl.pallas_call(
        fn,
        grid=(steps,),
        in_specs=in_specs,
        out_specs=pl.BlockSpec((_RB, d), lambda i: (blk0 + i, 0)),
        out_shape=jax.ShapeDtypeStruct((b_total, d), jnp.float32),
        **kwargs,
    )(*ins)


def kernel(x, table):
    (b,) = x.shape
    _, d = table.shape
    x = x.astype(jnp.int32)
    p, q = _pe_tables(d)
    out = None
    row0 = 0
    for bs in _SLICES:
        xj = lax.slice(x, (row0,), (row0 + bs,))
        gj = _sc_gather_slice(xj, table)
        out = _tc_combine_slice(gj, p, q, out, row0, b)
        row0 += bs
    return out


# speedup vs baseline: 1.0762x; 1.0762x over previous
import functools
import math

import jax
import jax.numpy as jnp
from jax import lax
from jax.experimental import pallas as pl
from jax.experimental.pallas import tpu as pltpu
from jax.experimental.pallas import tpu_sc as plsc

_RB = 256
_SLICES = (2048, 2048, 2048, 1536, 512)


def _sc_gather_slice(x_slice, table):
    (b,) = x_slice.shape
    _, d = table.shape
    info = plsc.get_sparse_core_info()
    nc, ns = info.num_cores, info.num_subcores
    nw = nc * ns
    b_per_w = b // nw
    k = 8
    n_chunks = b_per_w // k
    nbuf = min(6, n_chunks)

    mesh = plsc.VectorSubcoreMesh(core_axis_name="c", subcore_axis_name="s")

    @functools.partial(
        pl.kernel,
        mesh=mesh,
        out_type=jax.ShapeDtypeStruct((b, d), jnp.float32),
        scratch_types=[
            pltpu.VMEM((b_per_w,), jnp.int32),
            *[pltpu.VMEM((k, d), jnp.float32) for _ in range(nbuf)],
            *[pltpu.SemaphoreType.DMA for _ in range(nbuf)],
            *[pltpu.SemaphoreType.DMA for _ in range(nbuf)],
        ],
    )
    def gather_kernel(idx_hbm, table_hbm, out_hbm, idx_v, *rest):
        bufs = rest[:nbuf]
        gsems = rest[nbuf : 2 * nbuf]
        ssems = rest[2 * nbuf :]
        wid = lax.axis_index("s") * nc + lax.axis_index("c")
        base = wid * b_per_w
        pltpu.sync_copy(idx_hbm.at[pl.ds(base, b_per_w)], idx_v)
        gcp = [None] * nbuf
        scp = [None] * nbuf
        for c in range(nbuf):
            gcp[c] = pltpu.async_copy(
                table_hbm.at[idx_v.at[pl.ds(c * k, k)]], bufs[c], gsems[c]
            )
        for c in range(n_chunks):
            s = c % nbuf
            gcp[s].wait()
            scp[s] = pltpu.async_copy(
                bufs[s], out_hbm.at[pl.ds(base + c * k, k)], ssems[s]
            )
            nx = c + nbuf
            if nx < n_chunks:
                scp[s].wait()
                gcp[s] = pltpu.async_copy(
                    table_hbm.at[idx_v.at[pl.ds(nx * k, k)]], bufs[s], gsems[s]
                )
        for c in range(max(0, n_chunks - nbuf), n_chunks):
            scp[c % nbuf].wait()

    return gather_kernel(x_slice, table)


def _pe_tables(b, d):
    nl = -math.log(10000.0) / float(d)
    nblk = b // _RB

    def body(p_ref, q_ref, a_ref, b_ref):
        col = lax.broadcasted_iota(jnp.int32, (1, d), 1)
        w = jnp.exp((col - (col % 2)).astype(jnp.float32) * nl)
        r = lax.broadcasted_iota(jnp.int32, (_RB, 1), 0).astype(jnp.float32)
        ang = r * w
        p_ref[...] = jnp.cos(ang)
        q_ref[...] = jnp.sin(ang)
        blk = lax.broadcasted_iota(jnp.int32, (nblk, 1), 0).astype(jnp.float32)
        base_ang = (blk * float(_RB)) * w
        sb = jnp.sin(base_ang)
        cb = jnp.cos(base_ang)
        even = (col % 2) == 0
        a_ref[...] = jnp.where(even, sb, cb)
        b_ref[...] = jnp.where(even, cb, -sb)

    return pl.pallas_call(
        body,
        out_shape=(
            jax.ShapeDtypeStruct((_RB, d), jnp.float32),
            jax.ShapeDtypeStruct((_RB, d), jnp.float32),
            jax.ShapeDtypeStruct((nblk, d), jnp.float32),
            jax.ShapeDtypeStruct((nblk, d), jnp.float32),
        ),
    )()


def _tc_combine_slice(g, p, q, a, bv, prev_out, row0, b_total):
    bs, d = g.shape
    steps = bs // _RB
    blk0 = row0 // _RB
    scale = math.sqrt(float(d))

    def body(g_ref, p_ref, q_ref, a_ref, b_ref, *refs):
        o_ref = refs[-1]
        o_ref[...] = (
            g_ref[...] * scale + a_ref[0] * p_ref[...] + b_ref[0] * q_ref[...]
        )

    nblk_total = a.shape[0]
    ins = [g, p, q, a.reshape(nblk_total, 1, d), bv.reshape(nblk_total, 1, d)]
    in_specs = [
        pl.BlockSpec((_RB, d), lambda i: (i, 0)),
        pl.BlockSpec((_RB, d), lambda i: (0, 0)),
        pl.BlockSpec((_RB, d), lambda i: (0, 0)),
        pl.BlockSpec((1, 1, d), lambda i: (blk0 + i, 0, 0)),
        pl.BlockSpec((1, 1, d), lambda i: (blk0 + i, 0, 0)),
    ]
    kwargs = {}
    if prev_out is not None:
        ins.append(prev_out)
        in_specs.append(pl.BlockSpec(memory_space=pltpu.HBM))
        kwargs["input_output_aliases"] = {5: 0}

    return pl.pallas_call(
        body,
        grid=(steps,),
        in_specs=in_specs,
        out_specs=pl.BlockSpec((_RB, d), lambda i: (blk0 + i, 0)),
        out_shape=jax.ShapeDtypeStruct((b_total, d), jnp.float32),
        **kwargs,
    )(*ins)


def kernel(x, table):
    (b,) = x.shape
    _, d = table.shape
    x = x.astype(jnp.int32)
    p, q, a, bv = _pe_tables(b, d)
    out = None
    row0 = 0
    for bs in _SLICES:
        xj = lax.slice(x, (row0,), (row0 + bs,))
        gj = _sc_gather_slice(xj, table)
        out = _tc_combine_slice(gj, p, q, a, bv, out, row0, b)
        row0 += bs
    return out
